# baseline (speedup 1.0000x reference)
import jax
import jax.numpy as jnp
from jax.experimental import pallas as pl
from jax.experimental.pallas import tpu as pltpu

_B_BLK = 128


def _body(x_ref, states_ref, len_ref, acc_ref):
    i = pl.program_id(0)
    x = x_ref[...]
    states_ref[...] = jnp.transpose(x, (1, 0, 2))
    rows = jnp.sum(x, axis=2)
    nz = jnp.where(rows != 0.0, 1.0, 0.0)
    acc_ref[...] = jnp.sum(nz, axis=0)[None, :]
    len_ref[...] = acc_ref[...].astype(jnp.int32)


def kernel(batch):
    S, B, D = batch.shape
    states, lengths = pl.pallas_call(
        _body,
        grid=(B // _B_BLK,),
        in_specs=[pl.BlockSpec((S, _B_BLK, D), lambda i: (0, i, 0))],
        out_specs=[
            pl.BlockSpec((_B_BLK, S, D), lambda i: (i, 0, 0)),
            pl.BlockSpec((1, _B_BLK), lambda i: (0, i)),
        ],
        out_shape=[
            jax.ShapeDtypeStruct((B, S, D), batch.dtype),
            jax.ShapeDtypeStruct((1, B), jnp.int32),
        ],
        scratch_shapes=[pltpu.VMEM((1, _B_BLK), jnp.float32)],
        compiler_params=pltpu.CompilerParams(
            dimension_semantics=("parallel",),
        ),
    )(batch)
    return states, lengths.reshape(B)
